# Initial kernel scaffold; baseline (speedup 1.0000x reference)
import jax
import jax.numpy as jnp
from jax.experimental import pallas as pl


def kernel(doc, qry, table):
    raise NotImplementedError("write your pallas kernel here")


# speedup vs baseline: 5.2556x; 5.2556x over previous
import functools

import jax
import jax.numpy as jnp
from jax import lax
from jax.experimental import pallas as pl
from jax.experimental.pallas import tpu as pltpu
from jax.experimental.pallas import tpu_sc as plsc

D = 128
CH = 128


@functools.cache
def _build(n_doc, n_qry):
    info = plsc.get_sparse_core_info()
    nc, ns = info.num_cores, info.num_subcores
    nw = nc * ns
    doc_pw = n_doc // nw
    qry_pw = n_qry // nw
    doc_ch = doc_pw // CH
    qry_ch = qry_pw // CH
    mesh = plsc.VectorSubcoreMesh(core_axis_name="c", subcore_axis_name="s")

    @functools.partial(
        pl.kernel,
        out_type=(
            jax.ShapeDtypeStruct((n_doc, D), jnp.float32),
            jax.ShapeDtypeStruct((n_qry, D), jnp.float32),
        ),
        mesh=mesh,
        scratch_types=[
            pltpu.VMEM((doc_pw,), jnp.int32),
            pltpu.VMEM((qry_pw,), jnp.int32),
            pltpu.VMEM((CH, D), jnp.float32),
            pltpu.SemaphoreType.DMA,
        ],
    )
    def k(table, doc_idx, qry_idx, doc_out, qry_out, didx_v, qidx_v, rows_v,
          sem):
        wid = lax.axis_index("s") * nc + lax.axis_index("c")
        dbase = wid * doc_pw
        qbase = wid * qry_pw
        pltpu.sync_copy(doc_idx.at[pl.ds(dbase, doc_pw)], didx_v)
        pltpu.sync_copy(qry_idx.at[pl.ds(qbase, qry_pw)], qidx_v)

        def dbody(j, c):
            pltpu.async_copy(
                table.at[didx_v.at[pl.ds(j * CH, CH)]], rows_v, sem).wait()
            pltpu.sync_copy(rows_v, doc_out.at[pl.ds(dbase + j * CH, CH)])
            return c

        lax.fori_loop(0, doc_ch, dbody, 0)

        def qbody(j, c):
            pltpu.async_copy(
                table.at[qidx_v.at[pl.ds(j * CH, CH)]], rows_v, sem).wait()
            pltpu.sync_copy(rows_v, qry_out.at[pl.ds(qbase + j * CH, CH)])
            return c

        lax.fori_loop(0, qry_ch, qbody, 0)

    return k


def kernel(doc, qry, table):
    n_doc = doc.size
    n_qry = qry.size
    k = _build(n_doc, n_qry)
    doc_out, qry_out = k(table, doc.reshape(-1), qry.reshape(-1))
    return (doc_out.reshape(*doc.shape, D), qry_out.reshape(*qry.shape, D))


# speedup vs baseline: 7.0757x; 1.3463x over previous
import functools

import jax
import jax.numpy as jnp
from jax import lax
from jax.experimental import pallas as pl
from jax.experimental.pallas import tpu as pltpu
from jax.experimental.pallas import tpu_sc as plsc

D = 128
CH = 128
NBUF = 4


@functools.cache
def _build(n_doc, n_qry):
    info = plsc.get_sparse_core_info()
    nc, ns = info.num_cores, info.num_subcores
    nw = nc * ns
    doc_pw = n_doc // nw
    qry_pw = n_qry // nw
    doc_ch = doc_pw // CH
    qry_ch = qry_pw // CH
    mesh = plsc.VectorSubcoreMesh(core_axis_name="c", subcore_axis_name="s")

    @functools.partial(
        pl.kernel,
        out_type=(
            jax.ShapeDtypeStruct((n_doc, D), jnp.float32),
            jax.ShapeDtypeStruct((n_qry, D), jnp.float32),
        ),
        mesh=mesh,
        scratch_types=[
            pltpu.VMEM((doc_pw,), jnp.int32),
            pltpu.VMEM((qry_pw,), jnp.int32),
            pltpu.VMEM((NBUF, CH, D), jnp.float32),
            pltpu.SemaphoreType.DMA((NBUF,)),
            pltpu.SemaphoreType.DMA((NBUF,)),
        ],
    )
    def k(table, doc_idx, qry_idx, doc_out, qry_out, didx_v, qidx_v, rows_v,
          gsem, osem):
        wid = lax.axis_index("s") * nc + lax.axis_index("c")
        dbase = wid * doc_pw
        qbase = wid * qry_pw
        pltpu.sync_copy(doc_idx.at[pl.ds(dbase, doc_pw)], didx_v)
        pltpu.sync_copy(qry_idx.at[pl.ds(qbase, qry_pw)], qidx_v)

        def gather(idx_v, j, b):
            pltpu.async_copy(
                table.at[idx_v.at[pl.ds(j * CH, CH)]], rows_v.at[b],
                gsem.at[b])

        def wait_gather(b):
            pltpu.make_async_copy(
                table.at[pl.ds(0, CH)], rows_v.at[b], gsem.at[b]).wait()

        def put(out, base, j, b):
            pltpu.async_copy(
                rows_v.at[b], out.at[pl.ds(base + j * CH, CH)], osem.at[b])

        def wait_put(out, base, b):
            pltpu.make_async_copy(
                rows_v.at[b], out.at[pl.ds(base, CH)], osem.at[b]).wait()

        def run(idx_v, out, base, nch):
            ng = nch // NBUF
            for b in range(NBUF):
                gather(idx_v, b, b)

            def body(g, c):
                for b in range(NBUF):
                    wait_gather(b)
                    put(out, base, g * NBUF + b, b)

                @pl.when(g + 1 < ng)
                def _():
                    for b in range(NBUF):
                        wait_put(out, base, b)
                        gather(idx_v, (g + 1) * NBUF + b, b)

                @pl.when(g + 1 == ng)
                def _():
                    for b in range(NBUF):
                        wait_put(out, base, b)

                return c

            lax.fori_loop(0, ng, body, 0)

        run(didx_v, doc_out, dbase, doc_ch)
        run(qidx_v, qry_out, qbase, qry_ch)

    return k


def kernel(doc, qry, table):
    n_doc = doc.size
    n_qry = qry.size
    k = _build(n_doc, n_qry)
    doc_out, qry_out = k(table, doc.reshape(-1), qry.reshape(-1))
    return (doc_out.reshape(*doc.shape, D), qry_out.reshape(*qry.shape, D))


# speedup vs baseline: 7.6432x; 1.0802x over previous
import functools

import jax
import jax.numpy as jnp
from jax import lax
from jax.experimental import pallas as pl
from jax.experimental.pallas import tpu as pltpu
from jax.experimental.pallas import tpu_sc as plsc

D = 128
MAXCH = 128
NBUF = 4


@functools.cache
def _build(n_rows, doc_w, qry_w):
    info = plsc.get_sparse_core_info()
    nc, ns = info.num_cores, info.num_subcores
    nw = nc * ns
    rpw = n_rows // nw
    doc_parts = [(c, min(MAXCH, doc_w - c)) for c in range(0, doc_w, MAXCH)]
    doc_rg = NBUF // len(doc_parts)
    qry_rg = NBUF
    doc_ng = rpw // doc_rg
    qry_ng = rpw // qry_rg
    mesh = plsc.VectorSubcoreMesh(core_axis_name="c", subcore_axis_name="s")

    @functools.partial(
        pl.kernel,
        out_type=(
            jax.ShapeDtypeStruct((n_rows, doc_w, D), jnp.float32),
            jax.ShapeDtypeStruct((n_rows, qry_w, D), jnp.float32),
        ),
        mesh=mesh,
        scratch_types=[
            pltpu.VMEM((rpw, doc_w), jnp.int32),
            pltpu.VMEM((rpw, qry_w), jnp.int32),
            pltpu.VMEM((NBUF, MAXCH, D), jnp.float32),
            pltpu.SemaphoreType.DMA((NBUF,)),
            pltpu.SemaphoreType.DMA((NBUF,)),
        ],
    )
    def k(table, doc_idx, qry_idx, doc_out, qry_out, didx_v, qidx_v, rows_v,
          gsem, osem):
        wid = lax.axis_index("s") * nc + lax.axis_index("c")
        row0 = wid * rpw
        pltpu.sync_copy(doc_idx.at[pl.ds(row0, rpw)], didx_v)
        pltpu.sync_copy(qry_idx.at[pl.ds(row0, rpw)], qidx_v)

        doc_ops = [(i, c, n) for i in range(doc_rg) for (c, n) in doc_parts]
        qry_ops = [(b, 0, qry_w) for b in range(qry_rg)]

        def gather(idx_v, g, rg, b, op):
            i, c, n = op
            pltpu.async_copy(
                table.at[idx_v.at[g * rg + i, pl.ds(c, n)]],
                rows_v.at[b, pl.ds(0, n)], gsem.at[b])

        def wait_gather(b, op):
            n = op[2]
            src = table.at[pl.ds(0, n)] if n % 8 == 0 else qry_out.at[0]
            pltpu.make_async_copy(
                src, rows_v.at[b, pl.ds(0, n)], gsem.at[b]).wait()

        def put(out, g, rg, b, op):
            i, c, n = op
            pltpu.async_copy(
                rows_v.at[b, pl.ds(0, n)],
                out.at[row0 + g * rg + i, pl.ds(c, n)], osem.at[b])

        def wait_put(out, b, op):
            _, c, n = op
            pltpu.make_async_copy(
                rows_v.at[b, pl.ds(0, n)], out.at[0, pl.ds(c, n)],
                osem.at[b]).wait()

        def run(idx_v, out, rg, ng, ops):
            for b, op in enumerate(ops):
                gather(idx_v, 0, rg, b, op)

            def body(g, carry):
                for b, op in enumerate(ops):
                    wait_gather(b, op)
                    put(out, g, rg, b, op)

                @pl.when(g + 1 < ng)
                def _():
                    for b, op in enumerate(ops):
                        wait_put(out, b, op)
                        gather(idx_v, g + 1, rg, b, op)

                @pl.when(g + 1 == ng)
                def _():
                    for b, op in enumerate(ops):
                        wait_put(out, b, op)

                return carry

            lax.fori_loop(0, ng, body, 0)

        run(didx_v, doc_out, doc_rg, doc_ng, doc_ops)
        run(qidx_v, qry_out, qry_rg, qry_ng, qry_ops)

    return k


def kernel(doc, qry, table):
    k = _build(doc.shape[0], doc.shape[1], qry.shape[1])
    return k(table, doc, qry)


# speedup vs baseline: 8.0445x; 1.0525x over previous
import functools

import jax
import jax.numpy as jnp
from jax import lax
from jax.experimental import pallas as pl
from jax.experimental.pallas import tpu as pltpu
from jax.experimental.pallas import tpu_sc as plsc

D = 128
CH_BIG = 128
CH_SM = 72
NRING = 4


@functools.cache
def _build(n_rows, doc_w, qry_w):
    info = plsc.get_sparse_core_info()
    nc, ns = info.num_cores, info.num_subcores
    nw = nc * ns
    rpw = n_rows // nw
    nstage = 4
    rps = rpw // nstage
    doc_rg = NRING
    qry_rg = 2 * NRING
    doc_ng = rps // doc_rg
    qry_ng = rpw // qry_rg
    mesh = plsc.VectorSubcoreMesh(core_axis_name="c", subcore_axis_name="s")

    @functools.partial(
        pl.kernel,
        out_type=(
            jax.ShapeDtypeStruct((n_rows, doc_w, D), jnp.float32),
            jax.ShapeDtypeStruct((n_rows, qry_w, D), jnp.float32),
        ),
        mesh=mesh,
        scratch_types=[
            pltpu.VMEM((rps, doc_w), jnp.int32),
            pltpu.VMEM((rpw, qry_w), jnp.int32),
            pltpu.VMEM((NRING, CH_BIG, D), jnp.float32),
            pltpu.VMEM((NRING, CH_SM, D), jnp.float32),
            pltpu.SemaphoreType.DMA((2 * NRING,)),
            pltpu.SemaphoreType.DMA((2 * NRING,)),
        ],
    )
    def k(table, doc_idx, qry_idx, doc_out, qry_out, didx_v, qidx_v, big_v,
          sm_v, gsem, osem):
        wid = lax.axis_index("s") * nc + lax.axis_index("c")
        row0 = wid * rpw
        pltpu.sync_copy(qry_idx.at[pl.ds(row0, rpw)], qidx_v)

        doc_slots = []
        for i in range(doc_rg):
            doc_slots.append((i, 0, CH_BIG, big_v, i, i))
            doc_slots.append((i, CH_BIG, doc_w - CH_BIG, sm_v, i, NRING + i))
        qry_slots = []
        for i in range(qry_rg):
            buf = big_v if i < NRING else sm_v
            qry_slots.append((i, 0, qry_w, buf, i % NRING, i))

        def gather(idx_v, g, rg, slot):
            i, c, n, buf, bi, si = slot
            pltpu.async_copy(
                table.at[idx_v.at[g * rg + i, pl.ds(c, n)]],
                buf.at[bi, pl.ds(0, n)], gsem.at[si])

        def stage_doc(piece):
            pltpu.sync_copy(
                doc_idx.at[pl.ds(row0 + piece * rps, rps)], didx_v)

        def wait_gather(slot):
            _, c, n, buf, bi, si = slot
            src = table.at[pl.ds(0, n)] if n % 8 == 0 else qry_out.at[0]
            pltpu.make_async_copy(
                src, buf.at[bi, pl.ds(0, n)], gsem.at[si]).wait()

        def put(out, base, g, rg, slot):
            i, c, n, buf, bi, si = slot
            pltpu.async_copy(
                buf.at[bi, pl.ds(0, n)],
                out.at[base + g * rg + i, pl.ds(c, n)], osem.at[si])

        def wait_put(out, slot):
            _, c, n, buf, bi, si = slot
            pltpu.make_async_copy(
                buf.at[bi, pl.ds(0, n)], out.at[0, pl.ds(c, n)],
                osem.at[si]).wait()

        def run(idx_v, out, base, rg, ng, slots):
            for slot in slots:
                gather(idx_v, 0, rg, slot)

            def body(g, carry):
                for slot in slots:
                    wait_gather(slot)
                    put(out, base, g, rg, slot)

                @pl.when(g + 1 < ng)
                def _():
                    for slot in slots:
                        wait_put(out, slot)
                        gather(idx_v, g + 1, rg, slot)

                @pl.when(g + 1 == ng)
                def _():
                    for slot in slots:
                        wait_put(out, slot)

                return carry

            lax.fori_loop(0, ng, body, 0)

        for piece in range(nstage):
            stage_doc(piece)
            run(didx_v, doc_out, row0 + piece * rps, doc_rg, doc_ng,
                doc_slots)
        run(qidx_v, qry_out, row0, qry_rg, qry_ng, qry_slots)

    return k


def kernel(doc, qry, table):
    k = _build(doc.shape[0], doc.shape[1], qry.shape[1])
    return k(table, doc, qry)
